# baseline (speedup 1.0000x reference)
import functools

import jax
import jax.numpy as jnp
from jax import lax
from jax.experimental import pallas as pl
from jax.experimental.pallas import tpu as pltpu
from jax.experimental.pallas import tpu_sc as plsc

B, S, H, F, E, K = 2, 2048, 1024, 2688, 8, 2
T = B * S
A = T * K
AUX_W = 0.01

BR = 256
LOG_BR = 8
P_MAX = A + E * BR
NB = P_MAX // BR
NBP = 48

NW = 32
AS = A // NW
MYV = AS // 16
NVR = A // 16
TS = T // NW

_INTERP = False

_GDN = lax.GatherDimensionNumbers(
    offset_dims=(), collapsed_slice_dims=(0,), start_index_map=(0,))


def _perm(x, idx_vec):
    return lax.gather(x, idx_vec.reshape(16, 1), _GDN, (1,),
                      mode=lax.GatherScatterMode.PROMISE_IN_BOUNDS)


def _allsum(x, iota):
    for k in (1, 2, 4, 8):
        x = x + _perm(x, (iota + k) & 15)
    return x


def _prefix(x, iota):
    for k in (1, 2, 4, 8):
        x = x + jnp.where(iota >= k, _perm(x, jnp.maximum(iota - k, 0)), 0)
    return x


def _router_body(x_ref, wr_ref, idx_ref, val_ref, loss_ref):
    x = x_ref[...]
    wr = wr_ref[...]
    logits = jnp.dot(x, wr, preferred_element_type=jnp.float32)
    m = jnp.max(logits, axis=-1, keepdims=True)
    ex = jnp.exp(logits - m)
    p = ex / jnp.sum(ex, axis=-1, keepdims=True)

    iota = jax.lax.broadcasted_iota(jnp.int32, p.shape, 1)
    m1 = jnp.max(p, axis=-1, keepdims=True)
    i1 = jnp.min(jnp.where(p >= m1, iota, E), axis=-1, keepdims=True)
    pm = jnp.where(iota == i1, -1.0, p)
    m2 = jnp.max(pm, axis=-1, keepdims=True)
    i2 = jnp.min(jnp.where(pm >= m2, iota, E), axis=-1, keepdims=True)

    den = m1 + m2 + 1e-9
    idx_ref[...] = jnp.concatenate([i1, i2], axis=-1)
    val_ref[...] = jnp.concatenate([m1 / den, m2 / den], axis=-1)

    sum_p = jnp.sum(p, axis=0)
    load = ((iota == i1) | (iota == i2)).astype(jnp.float32)
    sum_load = jnp.sum(load, axis=0)
    loss = (E * AUX_W / (T * T)) * jnp.sum(sum_p * sum_load)
    loss_ref[...] = jnp.broadcast_to(loss, (1, 1))


def _router(x_flat, w_router):
    return pl.pallas_call(
        _router_body,
        out_shape=(
            jax.ShapeDtypeStruct((T, K), jnp.int32),
            jax.ShapeDtypeStruct((T, K), jnp.float32),
            jax.ShapeDtypeStruct((1, 1), jnp.float32),
        ),
        interpret=_INTERP,
    )(x_flat, w_router)


def _dispatch_body(idx_hbm, x32_hbm,
                   xs_hbm, pos_hbm, be_hbm, nu_hbm,
                   idx_v, pos_v, tok_v, rows_v, be_v, nu_v, sem):
    wid = lax.axis_index("s") * 2 + lax.axis_index("c")
    pltpu.sync_copy(idx_hbm, idx_v)
    iota = lax.iota(jnp.int32, 16)
    zero = jnp.zeros((16,), jnp.int32)
    my_lo = wid * MYV

    @pl.loop(0, NVR, init_carry=(zero,) * (2 * E))
    def hist_res(j, carry):
        accs = list(carry)
        v = idx_v[pl.ds(pl.multiple_of(j * 16, 16), 16)]
        pred = jnp.where(j < my_lo, 1, 0)
        for e in range(E):
            m = jnp.where(v == e, 1, 0)
            accs[e] = accs[e] + m
            accs[E + e] = accs[E + e] + m * pred
        return tuple(accs)

    accs = hist_res

    tot = zero
    run = zero
    for e in range(E):
        tot = tot + jnp.where(iota == e, _allsum(accs[e], iota), 0)
        run = run + jnp.where(iota == e, _allsum(accs[E + e], iota), 0)

    psz = ((tot + (BR - 1)) >> LOG_BR) << LOG_BR
    inc = _prefix(psz, iota)
    runpos = (inc - psz) + run

    for jj in range(MYV):
        v = idx_v[pl.ds(pl.multiple_of((my_lo + jj) * 16, 16), 16)]
        posj = zero
        for e in range(E):
            m = v == e
            mi = jnp.where(m, 1, 0)
            pc = _prefix(mi, iota)
            base_e = _perm(runpos, zero + e)
            posj = jnp.where(m, base_e + pc - 1, posj)
            runpos = runpos + jnp.where(iota == e, _allsum(mi, iota), 0)
        pos_v[jj // 8, pl.ds((jj % 8) * 16, 16)] = posj
        glob = (my_lo + jj) * 16 + iota
        tok_v[pl.ds(jj * 16, 16)] = glob >> 1

    pltpu.sync_copy(pos_v, pos_hbm.at[wid])

    for h in range(2):
        pltpu.async_copy(
            x32_hbm.at[tok_v.at[pl.ds(h * 128, 128)]], rows_v, sem).wait()
        pltpu.async_copy(rows_v, xs_hbm.at[pos_v.at[h]], sem).wait()

    @pl.when(wid == 0)
    def _():
        for c3 in range(NBP // 16):
            bs = (iota + c3 * 16) * BR
            r = zero
            for e in range(E):
                end_e = _perm(inc, zero + e)
                r = r + jnp.where(bs >= end_e, 1, 0)
            be_v[pl.ds(c3 * 16, 16)] = jnp.minimum(r, E - 1)
        total = _perm(inc, zero + (E - 1))
        nu_v[...] = total >> LOG_BR
        pltpu.sync_copy(be_v, be_hbm)
        pltpu.sync_copy(nu_v, nu_hbm)


def _dispatch(idx_flat, x32):
    mesh = plsc.VectorSubcoreMesh(core_axis_name="c", subcore_axis_name="s")
    fn = functools.partial(
        pl.kernel, mesh=mesh,
        out_type=[
            jax.ShapeDtypeStruct((P_MAX, H // 2), jnp.int32),
            jax.ShapeDtypeStruct((NW, 2, 128), jnp.int32),
            jax.ShapeDtypeStruct((NBP,), jnp.int32),
            jax.ShapeDtypeStruct((16,), jnp.int32),
        ],
        scratch_types=[
            pltpu.VMEM((A,), jnp.int32),
            pltpu.VMEM((2, 128), jnp.int32),
            pltpu.VMEM((AS,), jnp.int32),
            pltpu.VMEM((128, H // 2), jnp.int32),
            pltpu.VMEM((NBP,), jnp.int32),
            pltpu.VMEM((16,), jnp.int32),
            pltpu.SemaphoreType.DMA,
        ],
    )(_dispatch_body)
    return fn(idx_flat, x32)


def _gemm_body(be_ref, nu_ref, xs_ref, wg_ref, wu_ref, wd_ref, out_ref):
    b = pl.program_id(0)

    @pl.when(b < nu_ref[0])
    def _():
        xs = xs_ref[...]
        gate = jnp.dot(xs, wg_ref[0], preferred_element_type=jnp.float32)
        up = jnp.dot(xs, wu_ref[0], preferred_element_type=jnp.float32)
        g = (gate * jax.lax.logistic(gate) * up).astype(jnp.bfloat16)
        out_ref[...] = jnp.dot(g, wd_ref[0], preferred_element_type=jnp.float32)


def _gemm(x_sorted, block_expert, nused, w_gate, w_up, w_down):
    grid_spec = pltpu.PrefetchScalarGridSpec(
        num_scalar_prefetch=2,
        grid=(NB,),
        in_specs=[
            pl.BlockSpec((BR, H), lambda b, be, nu: (b, 0)),
            pl.BlockSpec((1, H, F), lambda b, be, nu: (be[b], 0, 0)),
            pl.BlockSpec((1, H, F), lambda b, be, nu: (be[b], 0, 0)),
            pl.BlockSpec((1, F, H), lambda b, be, nu: (be[b], 0, 0)),
        ],
        out_specs=pl.BlockSpec((BR, H), lambda b, be, nu: (b, 0)),
    )
    return pl.pallas_call(
        _gemm_body,
        grid_spec=grid_spec,
        out_shape=jax.ShapeDtypeStruct((P_MAX, H), jnp.float32),
        interpret=_INTERP,
    )(block_expert, nused, x_sorted,
      w_gate.astype(jnp.bfloat16), w_up.astype(jnp.bfloat16),
      w_down.astype(jnp.bfloat16))


def _combine_body(xo_hbm, pos_hbm, val_hbm, y_hbm,
                  posv, valv, rows_v, out_v, sem):
    wid = lax.axis_index("s") * 2 + lax.axis_index("c")
    iota = lax.iota(jnp.int32, 16)
    zero = jnp.zeros((16,), jnp.int32)
    pltpu.sync_copy(pos_hbm.at[wid], posv)
    pltpu.sync_copy(val_hbm.at[pl.ds(wid * AS, AS)], valv)

    @pl.loop(0, 16)
    def chunk(c):
        pltpu.async_copy(
            xo_hbm.at[posv.at[c // 8, pl.ds((c % 8) * 16, 16)]],
            rows_v, sem).wait()
        wv = valv[pl.ds(pl.multiple_of(c * 16, 16), 16)]
        for i in range(8):
            w0 = _perm(wv, zero + 2 * i)
            w1 = _perm(wv, zero + 2 * i + 1)
            for q in range(H // 16):
                sl = pl.ds(q * 16, 16)
                out_v[i, sl] = w0 * rows_v[2 * i, sl] + w1 * rows_v[2 * i + 1, sl]
        pltpu.sync_copy(out_v, y_hbm.at[pl.ds(wid * TS + c * 8, 8)])


def _combine(x_out, pos, val_flat):
    mesh = plsc.VectorSubcoreMesh(core_axis_name="c", subcore_axis_name="s")
    fn = functools.partial(
        pl.kernel, mesh=mesh,
        out_type=[jax.ShapeDtypeStruct((T, H), jnp.float32)],
        scratch_types=[
            pltpu.VMEM((2, 128), jnp.int32),
            pltpu.VMEM((AS,), jnp.float32),
            pltpu.VMEM((16, H), jnp.float32),
            pltpu.VMEM((8, H), jnp.float32),
            pltpu.SemaphoreType.DMA,
        ],
    )(_combine_body)
    return fn(x_out, pos, val_flat)


def kernel(x, w_router, w_gate, w_up, w_down):
    x_flat = x.reshape(T, H)
    x32 = lax.bitcast_convert_type(
        x_flat.astype(jnp.bfloat16).reshape(T, H // 2, 2), jnp.int32)
    idx, val, loss = _router(x_flat, w_router)

    xs32, pos, be, nu = _dispatch(idx.reshape(A), x32)

    xs16 = lax.bitcast_convert_type(xs32, jnp.bfloat16).reshape(P_MAX, H)
    x_out = _gemm(xs16, be[:NB], nu[:1], w_gate, w_up, w_down)

    (y,) = _combine(x_out, pos, val.reshape(A))
    return y.reshape(B, S, H), loss[0, 0]


# speedup vs baseline: 1.6695x; 1.6695x over previous
import functools

import jax
import jax.numpy as jnp
from jax import lax
from jax.experimental import pallas as pl
from jax.experimental.pallas import tpu as pltpu
from jax.experimental.pallas import tpu_sc as plsc

B, S, H, F, E, K = 2, 2048, 1024, 2688, 8, 2
T = B * S
A = T * K
AUX_W = 0.01

BR = 256
LOG_BR = 8
P_MAX = A + E * BR
NB = P_MAX // BR
NBP = 48

NW = 32
AS = A // NW
MYV = AS // 16
NVR = A // 16
TS = T // NW

_INTERP = False

_GDN = lax.GatherDimensionNumbers(
    offset_dims=(), collapsed_slice_dims=(0,), start_index_map=(0,))


def _perm(x, idx_vec):
    return lax.gather(x, idx_vec.reshape(16, 1), _GDN, (1,),
                      mode=lax.GatherScatterMode.PROMISE_IN_BOUNDS)


def _allsum(x, iota):
    for k in (1, 2, 4, 8):
        x = x + _perm(x, (iota + k) & 15)
    return x


def _prefix(x, iota):
    for k in (1, 2, 4, 8):
        x = x + jnp.where(iota >= k, _perm(x, jnp.maximum(iota - k, 0)), 0)
    return x


def _router_body(x_ref, wr_ref, idx_ref, val_ref, loss_ref):
    x = x_ref[...]
    wr = wr_ref[...]
    logits = jnp.dot(x, wr, preferred_element_type=jnp.float32)
    m = jnp.max(logits, axis=-1, keepdims=True)
    ex = jnp.exp(logits - m)
    p = ex / jnp.sum(ex, axis=-1, keepdims=True)

    iota = jax.lax.broadcasted_iota(jnp.int32, p.shape, 1)
    m1 = jnp.max(p, axis=-1, keepdims=True)
    i1 = jnp.min(jnp.where(p >= m1, iota, E), axis=-1, keepdims=True)
    pm = jnp.where(iota == i1, -1.0, p)
    m2 = jnp.max(pm, axis=-1, keepdims=True)
    i2 = jnp.min(jnp.where(pm >= m2, iota, E), axis=-1, keepdims=True)

    den = m1 + m2 + 1e-9
    idx_ref[...] = jnp.concatenate([i1, i2], axis=-1)
    val_ref[...] = jnp.concatenate([m1 / den, m2 / den], axis=-1)

    sum_p = jnp.sum(p, axis=0)
    load = ((iota == i1) | (iota == i2)).astype(jnp.float32)
    sum_load = jnp.sum(load, axis=0)
    loss = (E * AUX_W / (T * T)) * jnp.sum(sum_p * sum_load)
    loss_ref[...] = jnp.broadcast_to(loss, (1, 1))


def _router(x_flat, w_router):
    return pl.pallas_call(
        _router_body,
        out_shape=(
            jax.ShapeDtypeStruct((T, K), jnp.int32),
            jax.ShapeDtypeStruct((T, K), jnp.float32),
            jax.ShapeDtypeStruct((1, 1), jnp.float32),
        ),
        interpret=_INTERP,
    )(x_flat, w_router)


def _dispatch_body(idx_hbm, x_hbm,
                   xs_hbm, pos_hbm, be_hbm, nu_hbm,
                   idx_v, pos_v, tok_v, rows_v, be_v, nu_v, sem):
    wid = lax.axis_index("s") * 2 + lax.axis_index("c")
    pltpu.sync_copy(idx_hbm, idx_v)
    iota = lax.iota(jnp.int32, 16)
    zero = jnp.zeros((16,), jnp.int32)
    my_lo = wid * MYV

    @pl.loop(0, NVR, init_carry=(zero,) * (2 * E))
    def hist_res(j, carry):
        accs = list(carry)
        v = idx_v[pl.ds(pl.multiple_of(j * 16, 16), 16)]
        pred = jnp.where(j < my_lo, 1, 0)
        for e in range(E):
            m = jnp.where(v == e, 1, 0)
            accs[e] = accs[e] + m
            accs[E + e] = accs[E + e] + m * pred
        return tuple(accs)

    accs = hist_res

    tot = zero
    run = zero
    for e in range(E):
        tot = tot + jnp.where(iota == e, _allsum(accs[e], iota), 0)
        run = run + jnp.where(iota == e, _allsum(accs[E + e], iota), 0)

    psz = ((tot + (BR - 1)) >> LOG_BR) << LOG_BR
    inc = _prefix(psz, iota)
    runpos = (inc - psz) + run

    for jj in range(MYV):
        v = idx_v[pl.ds(pl.multiple_of((my_lo + jj) * 16, 16), 16)]
        posj = zero
        for e in range(E):
            m = v == e
            mi = jnp.where(m, 1, 0)
            pc = _prefix(mi, iota)
            base_e = _perm(runpos, zero + e)
            posj = jnp.where(m, base_e + pc - 1, posj)
            runpos = runpos + jnp.where(iota == e, _allsum(mi, iota), 0)
        pos_v[jj // 4, pl.ds((jj % 4) * 16, 16)] = posj
        glob = (my_lo + jj) * 16 + iota
        tok_v[pl.ds(jj * 16, 16)] = glob >> 1

    pltpu.sync_copy(pos_v, pos_hbm.at[wid])

    for h in range(4):
        pltpu.async_copy(
            x_hbm.at[tok_v.at[pl.ds(h * 64, 64)]], rows_v, sem).wait()
        pltpu.async_copy(rows_v, xs_hbm.at[pos_v.at[h]], sem).wait()

    @pl.when(wid == 0)
    def _():
        for c3 in range(NBP // 16):
            bs = (iota + c3 * 16) * BR
            r = zero
            for e in range(E):
                end_e = _perm(inc, zero + e)
                r = r + jnp.where(bs >= end_e, 1, 0)
            be_v[pl.ds(c3 * 16, 16)] = jnp.minimum(r, E - 1)
        total = _perm(inc, zero + (E - 1))
        nu_v[...] = total >> LOG_BR
        pltpu.sync_copy(be_v, be_hbm)
        pltpu.sync_copy(nu_v, nu_hbm)


def _dispatch(idx_flat, x_flat):
    mesh = plsc.VectorSubcoreMesh(core_axis_name="c", subcore_axis_name="s")
    fn = functools.partial(
        pl.kernel, mesh=mesh,
        out_type=[
            jax.ShapeDtypeStruct((P_MAX, H), jnp.float32),
            jax.ShapeDtypeStruct((NW, 4, 64), jnp.int32),
            jax.ShapeDtypeStruct((NBP,), jnp.int32),
            jax.ShapeDtypeStruct((16,), jnp.int32),
        ],
        scratch_types=[
            pltpu.VMEM((A,), jnp.int32),
            pltpu.VMEM((4, 64), jnp.int32),
            pltpu.VMEM((AS,), jnp.int32),
            pltpu.VMEM((64, H), jnp.float32),
            pltpu.VMEM((NBP,), jnp.int32),
            pltpu.VMEM((16,), jnp.int32),
            pltpu.SemaphoreType.DMA,
        ],
    )(_dispatch_body)
    return fn(idx_flat, x_flat)


def _gemm_body(be_ref, nu_ref, xs_ref, wg_ref, wu_ref, wd_ref, out_ref):
    b = pl.program_id(0)

    @pl.when(b < nu_ref[0])
    def _():
        xs = xs_ref[...].astype(jnp.bfloat16)
        gate = jnp.dot(xs, wg_ref[0], preferred_element_type=jnp.float32)
        up = jnp.dot(xs, wu_ref[0], preferred_element_type=jnp.float32)
        g = (gate * jax.lax.logistic(gate) * up).astype(jnp.bfloat16)
        out_ref[...] = jnp.dot(g, wd_ref[0], preferred_element_type=jnp.float32)


def _gemm(x_sorted, block_expert, nused, w_gate, w_up, w_down):
    grid_spec = pltpu.PrefetchScalarGridSpec(
        num_scalar_prefetch=2,
        grid=(NB,),
        in_specs=[
            pl.BlockSpec((BR, H), lambda b, be, nu: (b, 0)),
            pl.BlockSpec((1, H, F), lambda b, be, nu: (be[b], 0, 0)),
            pl.BlockSpec((1, H, F), lambda b, be, nu: (be[b], 0, 0)),
            pl.BlockSpec((1, F, H), lambda b, be, nu: (be[b], 0, 0)),
        ],
        out_specs=pl.BlockSpec((BR, H), lambda b, be, nu: (b, 0)),
    )
    return pl.pallas_call(
        _gemm_body,
        grid_spec=grid_spec,
        out_shape=jax.ShapeDtypeStruct((P_MAX, H), jnp.float32),
        interpret=_INTERP,
    )(block_expert, nused, x_sorted,
      w_gate.astype(jnp.bfloat16), w_up.astype(jnp.bfloat16),
      w_down.astype(jnp.bfloat16))


def _combine_body(xo_hbm, pos_hbm, val_hbm, y_hbm,
                  posv, valv, rows_v, out_v, sem):
    wid = lax.axis_index("s") * 2 + lax.axis_index("c")
    iota = lax.iota(jnp.int32, 16)
    zero = jnp.zeros((16,), jnp.int32)
    pltpu.sync_copy(pos_hbm.at[wid], posv)
    pltpu.sync_copy(val_hbm.at[pl.ds(wid * AS, AS)], valv)

    @pl.loop(0, 16)
    def chunk(c):
        pltpu.async_copy(
            xo_hbm.at[posv.at[c // 4, pl.ds((c % 4) * 16, 16)]],
            rows_v, sem).wait()
        wv = valv[pl.ds(pl.multiple_of(c * 16, 16), 16)]
        for i in range(8):
            w0 = _perm(wv, zero + 2 * i)
            w1 = _perm(wv, zero + 2 * i + 1)
            for q in range(H // 16):
                sl = pl.ds(q * 16, 16)
                out_v[i, sl] = w0 * rows_v[2 * i, sl] + w1 * rows_v[2 * i + 1, sl]
        pltpu.sync_copy(out_v, y_hbm.at[pl.ds(wid * TS + c * 8, 8)])


def _combine(x_out, pos, val_flat):
    mesh = plsc.VectorSubcoreMesh(core_axis_name="c", subcore_axis_name="s")
    fn = functools.partial(
        pl.kernel, mesh=mesh,
        out_type=[jax.ShapeDtypeStruct((T, H), jnp.float32)],
        scratch_types=[
            pltpu.VMEM((4, 64), jnp.int32),
            pltpu.VMEM((AS,), jnp.float32),
            pltpu.VMEM((16, H), jnp.float32),
            pltpu.VMEM((8, H), jnp.float32),
            pltpu.SemaphoreType.DMA,
        ],
    )(_combine_body)
    return fn(x_out, pos, val_flat)


def kernel(x, w_router, w_gate, w_up, w_down):
    x_flat = x.reshape(T, H)
    idx, val, loss = _router(x_flat, w_router)

    xs, pos, be, nu = _dispatch(idx.reshape(A), x_flat)

    x_out = _gemm(xs, be[:NB], nu[:1], w_gate, w_up, w_down)

    (y,) = _combine(x_out, pos, val.reshape(A))
    return y.reshape(B, S, H), loss[0, 0]


# speedup vs baseline: 1.7432x; 1.0441x over previous
import functools

import jax
import jax.numpy as jnp
from jax import lax
from jax.experimental import pallas as pl
from jax.experimental.pallas import tpu as pltpu
from jax.experimental.pallas import tpu_sc as plsc

B, S, H, F, E, K = 2, 2048, 1024, 2688, 8, 2
T = B * S
A = T * K
AUX_W = 0.01

BR = 256
LOG_BR = 8
P_MAX = A + E * BR
NB = P_MAX // BR
NBP = 48

NW = 32
AS = A // NW
MYV = AS // 16
NVR = A // 16
TS = T // NW

_INTERP = False

_GDN = lax.GatherDimensionNumbers(
    offset_dims=(), collapsed_slice_dims=(0,), start_index_map=(0,))


def _perm(x, idx_vec):
    return lax.gather(x, idx_vec.reshape(16, 1), _GDN, (1,),
                      mode=lax.GatherScatterMode.PROMISE_IN_BOUNDS)


def _allsum(x, iota):
    for k in (1, 2, 4, 8):
        x = x + _perm(x, (iota + k) & 15)
    return x


def _prefix(x, iota):
    for k in (1, 2, 4, 8):
        x = x + jnp.where(iota >= k, _perm(x, jnp.maximum(iota - k, 0)), 0)
    return x


def _router_body(x_ref, wr_ref, idx_ref, val_ref, loss_ref):
    x = x_ref[...]
    wr = wr_ref[...]
    logits = jnp.dot(x, wr, preferred_element_type=jnp.float32)
    m = jnp.max(logits, axis=-1, keepdims=True)
    ex = jnp.exp(logits - m)
    p = ex / jnp.sum(ex, axis=-1, keepdims=True)

    iota = jax.lax.broadcasted_iota(jnp.int32, p.shape, 1)
    m1 = jnp.max(p, axis=-1, keepdims=True)
    i1 = jnp.min(jnp.where(p >= m1, iota, E), axis=-1, keepdims=True)
    pm = jnp.where(iota == i1, -1.0, p)
    m2 = jnp.max(pm, axis=-1, keepdims=True)
    i2 = jnp.min(jnp.where(pm >= m2, iota, E), axis=-1, keepdims=True)

    den = m1 + m2 + 1e-9
    idx_ref[...] = jnp.concatenate([i1, i2], axis=-1)
    val_ref[...] = jnp.concatenate([m1 / den, m2 / den], axis=-1)

    sum_p = jnp.sum(p, axis=0)
    load = ((iota == i1) | (iota == i2)).astype(jnp.float32)
    sum_load = jnp.sum(load, axis=0)
    loss = (E * AUX_W / (T * T)) * jnp.sum(sum_p * sum_load)
    loss_ref[...] = jnp.broadcast_to(loss, (1, 1))


def _router(x_flat, w_router):
    return pl.pallas_call(
        _router_body,
        out_shape=(
            jax.ShapeDtypeStruct((T, K), jnp.int32),
            jax.ShapeDtypeStruct((T, K), jnp.float32),
            jax.ShapeDtypeStruct((1, 1), jnp.float32),
        ),
        interpret=_INTERP,
    )(x_flat, w_router)


def _dispatch_body(idx_hbm, x_hbm,
                   xs_hbm, pos_hbm, be_hbm, nu_hbm,
                   idx_v, pos_v, tok_v, rows_a, rows_b, be_v, nu_v,
                   gsem_a, gsem_b, ssem):
    wid = lax.axis_index("s") * 2 + lax.axis_index("c")
    pltpu.sync_copy(idx_hbm, idx_v)
    iota = lax.iota(jnp.int32, 16)
    zero = jnp.zeros((16,), jnp.int32)
    my_lo = wid * MYV

    @pl.loop(0, NVR, init_carry=(zero,) * (2 * E))
    def hist_res(j, carry):
        accs = list(carry)
        v = idx_v[pl.ds(pl.multiple_of(j * 16, 16), 16)]
        pred = jnp.where(j < my_lo, 1, 0)
        for e in range(E):
            m = jnp.where(v == e, 1, 0)
            accs[e] = accs[e] + m
            accs[E + e] = accs[E + e] + m * pred
        return tuple(accs)

    accs = hist_res

    tot = zero
    run = zero
    for e in range(E):
        tot = tot + jnp.where(iota == e, _allsum(accs[e], iota), 0)
        run = run + jnp.where(iota == e, _allsum(accs[E + e], iota), 0)

    psz = ((tot + (BR - 1)) >> LOG_BR) << LOG_BR
    inc = _prefix(psz, iota)
    runpos = (inc - psz) + run

    for jj in range(MYV):
        v = idx_v[pl.ds(pl.multiple_of((my_lo + jj) * 16, 16), 16)]
        posj = zero
        for e in range(E):
            m = v == e
            mi = jnp.where(m, 1, 0)
            pc = _prefix(mi, iota)
            base_e = _perm(runpos, zero + e)
            posj = jnp.where(m, base_e + pc - 1, posj)
            runpos = runpos + jnp.where(iota == e, _allsum(mi, iota), 0)
        pos_v[jj // 2, pl.ds((jj % 2) * 16, 16)] = posj
        glob = (my_lo + jj) * 16 + iota
        tok_v[pl.ds(jj * 16, 16)] = glob >> 1

    pltpu.sync_copy(pos_v, pos_hbm.at[wid])

    bufs = (rows_a, rows_b)
    gsems = (gsem_a, gsem_b)
    pltpu.async_copy(x_hbm.at[tok_v.at[pl.ds(0, 32)]], rows_a, gsem_a)
    for c in range(8):
        buf = bufs[c % 2]
        pltpu.make_async_copy(x_hbm.at[tok_v.at[pl.ds(c * 32, 32)]],
                              buf, gsems[c % 2]).wait()
        if c + 1 < 8:
            nbuf = bufs[(c + 1) % 2]
            if c + 1 >= 2:
                pltpu.make_async_copy(nbuf, xs_hbm.at[pos_v.at[c - 1]],
                                      ssem).wait()
            pltpu.async_copy(
                x_hbm.at[tok_v.at[pl.ds((c + 1) * 32, 32)]], nbuf,
                gsems[(c + 1) % 2])
        pltpu.async_copy(buf, xs_hbm.at[pos_v.at[c]], ssem)
    pltpu.make_async_copy(rows_a, xs_hbm.at[pos_v.at[6]], ssem).wait()
    pltpu.make_async_copy(rows_b, xs_hbm.at[pos_v.at[7]], ssem).wait()

    @pl.when(wid == 0)
    def _():
        for c3 in range(NBP // 16):
            bs = (iota + c3 * 16) * BR
            r = zero
            for e in range(E):
                end_e = _perm(inc, zero + e)
                r = r + jnp.where(bs >= end_e, 1, 0)
            be_v[pl.ds(c3 * 16, 16)] = jnp.minimum(r, E - 1)
        total = _perm(inc, zero + (E - 1))
        nu_v[...] = total >> LOG_BR
        pltpu.sync_copy(be_v, be_hbm)
        pltpu.sync_copy(nu_v, nu_hbm)


def _dispatch(idx_flat, x_flat):
    mesh = plsc.VectorSubcoreMesh(core_axis_name="c", subcore_axis_name="s")
    fn = functools.partial(
        pl.kernel, mesh=mesh,
        out_type=[
            jax.ShapeDtypeStruct((P_MAX, H), jnp.float32),
            jax.ShapeDtypeStruct((NW, 8, 32), jnp.int32),
            jax.ShapeDtypeStruct((NBP,), jnp.int32),
            jax.ShapeDtypeStruct((16,), jnp.int32),
        ],
        scratch_types=[
            pltpu.VMEM((A,), jnp.int32),
            pltpu.VMEM((8, 32), jnp.int32),
            pltpu.VMEM((AS,), jnp.int32),
            pltpu.VMEM((32, H), jnp.float32),
            pltpu.VMEM((32, H), jnp.float32),
            pltpu.VMEM((NBP,), jnp.int32),
            pltpu.VMEM((16,), jnp.int32),
            pltpu.SemaphoreType.DMA,
            pltpu.SemaphoreType.DMA,
            pltpu.SemaphoreType.DMA,
        ],
    )(_dispatch_body)
    return fn(idx_flat, x_flat)


def _gemm_body(be_ref, nu_ref, xs_ref, wg_ref, wu_ref, wd_ref, out_ref):
    b = pl.program_id(0)

    @pl.when(b < nu_ref[0])
    def _():
        xs = xs_ref[...].astype(jnp.bfloat16)
        gate = jnp.dot(xs, wg_ref[0], preferred_element_type=jnp.float32)
        up = jnp.dot(xs, wu_ref[0], preferred_element_type=jnp.float32)
        g = (gate * jax.lax.logistic(gate) * up).astype(jnp.bfloat16)
        out_ref[...] = jnp.dot(g, wd_ref[0], preferred_element_type=jnp.float32)


def _gemm(x_sorted, block_expert, nused, w_gate, w_up, w_down):
    grid_spec = pltpu.PrefetchScalarGridSpec(
        num_scalar_prefetch=2,
        grid=(NB,),
        in_specs=[
            pl.BlockSpec((BR, H), lambda b, be, nu: (b, 0)),
            pl.BlockSpec((1, H, F), lambda b, be, nu: (be[b], 0, 0)),
            pl.BlockSpec((1, H, F), lambda b, be, nu: (be[b], 0, 0)),
            pl.BlockSpec((1, F, H), lambda b, be, nu: (be[b], 0, 0)),
        ],
        out_specs=pl.BlockSpec((BR, H), lambda b, be, nu: (b, 0)),
    )
    return pl.pallas_call(
        _gemm_body,
        grid_spec=grid_spec,
        out_shape=jax.ShapeDtypeStruct((P_MAX, H), jnp.float32),
        interpret=_INTERP,
    )(block_expert, nused, x_sorted,
      w_gate.astype(jnp.bfloat16), w_up.astype(jnp.bfloat16),
      w_down.astype(jnp.bfloat16))


def _combine_body(xo_hbm, pos_hbm, val_hbm, y_hbm,
                  posv, valv, rows_a, rows_b, out_v, sem_a, sem_b):
    wid = lax.axis_index("s") * 2 + lax.axis_index("c")
    iota = lax.iota(jnp.int32, 16)
    zero = jnp.zeros((16,), jnp.int32)
    pltpu.sync_copy(pos_hbm.at[wid], posv)
    pltpu.sync_copy(val_hbm.at[pl.ds(wid * AS, AS)], valv)

    def _idx(c):
        return posv.at[c // 2, pl.ds((c % 2) * 16, 16)]

    def _compute(c, rows_v):
        wv = valv[pl.ds(pl.multiple_of(c * 16, 16), 16)]
        for i in range(8):
            w0 = _perm(wv, zero + 2 * i)
            w1 = _perm(wv, zero + 2 * i + 1)
            for q in range(H // 16):
                sl = pl.ds(q * 16, 16)
                out_v[i, sl] = (w0 * rows_v[2 * i, sl]
                                + w1 * rows_v[2 * i + 1, sl])
        pltpu.sync_copy(out_v, y_hbm.at[pl.ds(wid * TS + c * 8, 8)])

    pltpu.async_copy(xo_hbm.at[_idx(0)], rows_a, sem_a)

    @pl.loop(0, 8)
    def _ring(g):
        c0 = g * 2
        pltpu.make_async_copy(xo_hbm.at[_idx(c0)], rows_a, sem_a).wait()
        pltpu.async_copy(xo_hbm.at[_idx(c0 + 1)], rows_b, sem_b)
        _compute(c0, rows_a)
        pltpu.make_async_copy(xo_hbm.at[_idx(c0 + 1)], rows_b, sem_b).wait()

        @pl.when(g < 7)
        def _():
            pltpu.async_copy(xo_hbm.at[_idx(c0 + 2)], rows_a, sem_a)

        _compute(c0 + 1, rows_b)


def _combine(x_out, pos, val_flat):
    mesh = plsc.VectorSubcoreMesh(core_axis_name="c", subcore_axis_name="s")
    fn = functools.partial(
        pl.kernel, mesh=mesh,
        out_type=[jax.ShapeDtypeStruct((T, H), jnp.float32)],
        scratch_types=[
            pltpu.VMEM((8, 32), jnp.int32),
            pltpu.VMEM((AS,), jnp.float32),
            pltpu.VMEM((16, H), jnp.float32),
            pltpu.VMEM((16, H), jnp.float32),
            pltpu.VMEM((8, H), jnp.float32),
            pltpu.SemaphoreType.DMA,
            pltpu.SemaphoreType.DMA,
        ],
    )(_combine_body)
    return fn(x_out, pos, val_flat)


def kernel(x, w_router, w_gate, w_up, w_down):
    x_flat = x.reshape(T, H)
    idx, val, loss = _router(x_flat, w_router)

    xs, pos, be, nu = _dispatch(idx.reshape(A), x_flat)

    x_out = _gemm(xs, be[:NB], nu[:1], w_gate, w_up, w_down)

    (y,) = _combine(x_out, pos, val.reshape(A))
    return y.reshape(B, S, H), loss[0, 0]
